# baseline (speedup 1.0000x reference)
import functools

import jax
import jax.numpy as jnp
from jax import lax
from jax.experimental import pallas as pl
from jax.experimental.pallas import tpu as pltpu
from jax.experimental.pallas import tpu_sc as plsc

M, D, KD, B, Q = 100000, 128, 64, 1024, 1024
K = 16
TILE = 2048
NCOL = M + B
NTILES = (NCOL + TILE - 1) // TILE
PCOL = NTILES * TILE
G = 16
NG = PCOL // G
GPT = TILE // G
NCAND = K * G
AUG = 128
NPAIRS = K // 2
PROWS = NCAND // 2
NEG = -1e30
NEG_INIT = -3e38

SC_NC, SC_NS = 2, 16
SC_NW = SC_NC * SC_NS

QBLK = 16


def _stage_a_body(q_ref, k_ref, w_ref, b_ref, gm_ref):
    s = lax.dot_general(q_ref[...], k_ref[...], (((1,), (1,)), ((), ())),
                        precision=lax.Precision.DEFAULT)
    s = s * w_ref[0, 0, :][None, :] + b_ref[0, 0, :][None, :]
    gm = s[:, 0:GPT]
    for j in range(1, G):
        gm = jnp.maximum(gm, s[:, j * GPT:(j + 1) * GPT])
    gm_ref[...] = gm


def _stage_a_call(query_keys, keys_ext, wcol3, bias3):
    return pl.pallas_call(
        _stage_a_body,
        grid=(NTILES,),
        in_specs=[
            pl.BlockSpec((Q, KD), lambda t: (0, 0)),
            pl.BlockSpec((TILE, KD), lambda t: (t, 0)),
            pl.BlockSpec((1, 1, TILE), lambda t: (t, 0, 0)),
            pl.BlockSpec((1, 1, TILE), lambda t: (t, 0, 0)),
        ],
        out_specs=pl.BlockSpec((Q, GPT), lambda t: (0, t)),
        out_shape=jax.ShapeDtypeStruct((Q, NG), jnp.float32),
    )(query_keys, keys_ext, wcol3, bias3)


def _stage_b_body(gm_ref, cand_ref, pair_ref):
    gmall = gm_ref[...]
    gid = lax.broadcasted_iota(jnp.int32, (QBLK, NG), 1)
    sels = []
    for _ in range(K):
        m = jnp.max(gmall, axis=1, keepdims=True)
        am = jnp.min(jnp.where(gmall == m, gid, NG), axis=1, keepdims=True)
        gmall = jnp.where(gid == am, NEG_INIT, gmall)
        sels.append(am)
    gsel = jnp.concatenate(sels, axis=1)
    tb = (gsel // GPT) * TILE + (gsel % GPT)
    ei = lax.broadcasted_iota(jnp.int32, (QBLK, 2, K, NPAIRS), 1)
    ji = lax.broadcasted_iota(jnp.int32, (QBLK, 2, K, NPAIRS), 3)
    cand_ref[...] = tb[:, None, :, None] + (2 * ji + ei) * GPT
    pb = (gsel // GPT) * (TILE // 2) + (gsel % GPT)
    pair_ref[...] = pb[:, :, None] + ji[:, 0, :, :] * GPT


def _stage_b_call(gm):
    return pl.pallas_call(
        _stage_b_body,
        grid=(Q // QBLK,),
        in_specs=[pl.BlockSpec((QBLK, NG), lambda i: (i, 0))],
        out_specs=[
            pl.BlockSpec((QBLK, 2, K, NPAIRS), lambda i: (i, 0, 0, 0)),
            pl.BlockSpec((QBLK, K, NPAIRS), lambda i: (i, 0, 0)),
        ],
        out_shape=[
            jax.ShapeDtypeStruct((Q, 2, K, NPAIRS), jnp.int32),
            jax.ShapeDtypeStruct((Q, K, NPAIRS), jnp.int32),
        ],
    )(gm)


def _sc_gather(table, idx_flat, nrows, width, chunk):
    mesh = plsc.VectorSubcoreMesh(core_axis_name="c", subcore_axis_name="s")
    per_w = nrows // SC_NW
    nchunk = per_w // chunk

    @functools.partial(
        pl.kernel, mesh=mesh,
        out_type=jax.ShapeDtypeStruct((nrows, width), jnp.float32),
        scratch_types=[
            pltpu.VMEM((chunk,), jnp.int32),
            pltpu.VMEM((chunk, width), jnp.float32),
            pltpu.SemaphoreType.DMA,
        ],
    )
    def k(table_hbm, idx_hbm, out_hbm, idx_v, rows_v, sem):
        wid = lax.axis_index("s") * SC_NC + lax.axis_index("c")
        for cc in range(nchunk):
            base = wid * per_w + cc * chunk
            pltpu.sync_copy(idx_hbm.at[pl.ds(base, chunk)], idx_v)
            pltpu.async_copy(table_hbm.at[idx_v], rows_v, sem).wait()
            pltpu.sync_copy(rows_v, out_hbm.at[pl.ds(base, chunk)])

    return k(table, idx_flat)


def _sc_gather_elems(table, idx_flat, nelem):
    mesh = plsc.VectorSubcoreMesh(core_axis_name="c", subcore_axis_name="s")
    per_w = nelem // SC_NW

    @functools.partial(
        pl.kernel, mesh=mesh,
        out_type=jax.ShapeDtypeStruct((nelem,), jnp.float32),
        scratch_types=[
            pltpu.VMEM((per_w,), jnp.int32),
            pltpu.VMEM((per_w,), jnp.float32),
            pltpu.SemaphoreType.DMA,
        ],
    )
    def k(table_hbm, idx_hbm, out_hbm, idx_v, out_v, sem):
        wid = lax.axis_index("s") * SC_NC + lax.axis_index("c")
        base = wid * per_w
        pltpu.sync_copy(idx_hbm.at[pl.ds(base, per_w)], idx_v)
        pltpu.async_copy(table_hbm.at[idx_v], out_v, sem).wait()
        pltpu.sync_copy(out_v, out_hbm.at[pl.ds(base, per_w)])

    return k(table, idx_flat)


def _stage_d_body(rows_ref, wc_ref, q_ref, cand_ref, attn_ref, idx_ref):
    rows = rows_ref[...]
    qv = q_ref[...]
    qi = lax.broadcasted_iota(jnp.int32, (QBLK, QBLK, PROWS), 0)
    oi = lax.broadcasted_iota(jnp.int32, (QBLK, QBLK, PROWS), 1)
    halves = []
    for e in range(2):
        kc = rows[:, e * KD:(e + 1) * KD]
        s_all = lax.dot_general(qv, kc, (((1,), (1,)), ((), ())),
                                precision=lax.Precision.DEFAULT)
        s3 = s_all.reshape(QBLK, QBLK, PROWS)
        halves.append(jnp.sum(jnp.where(qi == oi, s3, 0.0), axis=1))
    s = jnp.concatenate(halves, axis=1)

    w = wc_ref[...]
    s = s * w + jnp.where(w == 0.0, NEG, 0.0)

    cand = cand_ref[...]
    sid = lax.broadcasted_iota(jnp.int32, (QBLK, NCAND), 1)
    vs, gs = [], []
    for _ in range(K):
        m = jnp.max(s, axis=1, keepdims=True)
        am = jnp.min(jnp.where(s == m, sid, NCAND), axis=1, keepdims=True)
        gi = jnp.max(jnp.where(sid == am, cand, -1), axis=1, keepdims=True)
        s = jnp.where(sid == am, NEG_INIT, s)
        vs.append(m)
        gs.append(gi)
    vals = jnp.concatenate(vs, axis=1)
    mm = jnp.max(vals, axis=1, keepdims=True)
    e = jnp.exp(vals - mm)
    attn_ref[...] = e / jnp.sum(e, axis=1, keepdims=True)
    idx_ref[...] = jnp.concatenate(gs, axis=1)


def _stage_d_call(rows, w_cand, query_keys, cand_idx):
    nblk = Q // QBLK
    return pl.pallas_call(
        _stage_d_body,
        grid=(nblk,),
        in_specs=[
            pl.BlockSpec((QBLK * PROWS, AUG), lambda i: (i, 0)),
            pl.BlockSpec((QBLK, NCAND), lambda i: (i, 0)),
            pl.BlockSpec((QBLK, KD), lambda i: (i, 0)),
            pl.BlockSpec((QBLK, NCAND), lambda i: (i, 0)),
        ],
        out_specs=[
            pl.BlockSpec((QBLK, K), lambda i: (i, 0)),
            pl.BlockSpec((QBLK, K), lambda i: (i, 0)),
        ],
        out_shape=[
            jax.ShapeDtypeStruct((Q, K), jnp.float32),
            jax.ShapeDtypeStruct((Q, K), jnp.int32),
        ],
    )(rows, w_cand, query_keys, cand_idx)


def _wsum_body(rows_ref, attn_ref, out_ref):
    attn = attn_ref[...]
    acc = attn[:, 0:1] * rows_ref[pl.ds(0, Q), :]
    for kk in range(1, K):
        acc = acc + attn[:, kk:kk + 1] * rows_ref[pl.ds(kk * Q, Q), :]
    out_ref[...] = acc


def _wsum_call(rows, attn):
    return pl.pallas_call(
        _wsum_body,
        out_shape=jax.ShapeDtypeStruct((Q, D), jnp.float32),
    )(rows, attn)


def kernel(memory_bank, temporal_keys, access_counts, last_updated,
           attention_weights, new_patterns, new_keys, query_keys,
           current_time, num_retrievals):
    ct = current_time[0]
    decay = jnp.exp(-(ct - last_updated) * 0.1)
    ti = 0.4 * decay + 0.3 * jnp.log1p(access_counts) + 0.3 * attention_weights
    _, repl_idx = lax.top_k(-ti, B)
    repl_mask = jnp.zeros((M,), jnp.float32).at[repl_idx].set(1.0)
    c_new = 0.4 + 0.3 * jnp.log1p(1.0) + 0.3

    wcol = jnp.concatenate([ti, jnp.full((B,), c_new, jnp.float32)])
    bias = jnp.concatenate([repl_mask * NEG, jnp.zeros((B,), jnp.float32)])
    pad = PCOL - NCOL
    wcol = jnp.pad(wcol, (0, pad))
    bias = jnp.pad(bias, (0, pad), constant_values=NEG)
    keys_ext = jnp.pad(jnp.concatenate([temporal_keys, new_keys], axis=0),
                       ((0, pad), (0, 0)))

    gm = _stage_a_call(query_keys, keys_ext,
                       wcol.reshape(NTILES, 1, TILE),
                       bias.reshape(NTILES, 1, TILE))
    cand, pair = _stage_b_call(gm)
    cand_idx = cand.reshape(Q, NCAND)

    ktab2 = jnp.transpose(
        keys_ext.reshape(NTILES, NPAIRS, 2, GPT, KD),
        (0, 1, 3, 2, 4)).reshape(PCOL // 2, 2 * KD)
    crows = _sc_gather(ktab2, pair.reshape(-1), Q * PROWS, AUG, 512)

    wm = jnp.where(bias < 0.0, 0.0, wcol)
    w_cand = _sc_gather_elems(wm, cand_idx.reshape(-1),
                              Q * NCAND).reshape(Q, NCAND)

    attn, idx16 = _stage_d_call(crows, w_cand, query_keys, cand_idx)

    bank_ext = jnp.concatenate([memory_bank, new_patterns], axis=0)
    idx_flat = idx16.T.reshape(-1)
    rows = _sc_gather(bank_ext, idx_flat, Q * K, D, 512)
    out = _wsum_call(rows, attn)
    del num_retrievals
    return out


# speedup vs baseline: 1.4658x; 1.4658x over previous
import functools

import jax
import jax.numpy as jnp
from jax import lax
from jax.experimental import pallas as pl
from jax.experimental.pallas import tpu as pltpu
from jax.experimental.pallas import tpu_sc as plsc

M, D, KD, B, Q = 100000, 128, 64, 1024, 1024
K = 16
TILE = 2048
NCOL = M + B
NTILES = (NCOL + TILE - 1) // TILE
PCOL = NTILES * TILE
G = 16
NG = PCOL // G
GPT = TILE // G
NCAND = K * G
AUG = 128
NPAIRS = K // 2
PROWS = NCAND // 2
NEG = -1e30
NEG_INIT = -3e38

SC_NC, SC_NS = 2, 16
SC_NW = SC_NC * SC_NS

QBLK = 64


def _stage_a_body(q_ref, k_ref, w_ref, b_ref, gm_ref):
    s = lax.dot_general(q_ref[...], k_ref[...], (((1,), (1,)), ((), ())),
                        precision=lax.Precision.DEFAULT)
    s = s * w_ref[0, 0, :][None, :] + b_ref[0, 0, :][None, :]
    gm = s[:, 0:GPT]
    for j in range(1, G):
        gm = jnp.maximum(gm, s[:, j * GPT:(j + 1) * GPT])
    gm_ref[...] = gm


def _stage_a_call(query_keys, keys_ext, wcol3, bias3):
    return pl.pallas_call(
        _stage_a_body,
        grid=(NTILES,),
        in_specs=[
            pl.BlockSpec((Q, KD), lambda t: (0, 0)),
            pl.BlockSpec((TILE, KD), lambda t: (t, 0)),
            pl.BlockSpec((1, 1, TILE), lambda t: (t, 0, 0)),
            pl.BlockSpec((1, 1, TILE), lambda t: (t, 0, 0)),
        ],
        out_specs=pl.BlockSpec((Q, GPT), lambda t: (0, t)),
        out_shape=jax.ShapeDtypeStruct((Q, NG), jnp.float32),
    )(query_keys, keys_ext, wcol3, bias3)


def _stage_b_body(gm_ref, cand_ref, pair_ref):
    gmall = gm_ref[...]
    gid = lax.broadcasted_iota(jnp.int32, (QBLK, NG), 1)
    sels = []
    for _ in range(K):
        m = jnp.max(gmall, axis=1, keepdims=True)
        am = jnp.min(jnp.where(gmall == m, gid, NG), axis=1, keepdims=True)
        gmall = jnp.where(gid == am, NEG_INIT, gmall)
        sels.append(am)
    gsel = jnp.concatenate(sels, axis=1)
    tb = (gsel // GPT) * TILE + (gsel % GPT)
    ei = lax.broadcasted_iota(jnp.int32, (QBLK, 2, K, NPAIRS), 1)
    ji = lax.broadcasted_iota(jnp.int32, (QBLK, 2, K, NPAIRS), 3)
    cand_ref[...] = tb[:, None, :, None] + (2 * ji + ei) * GPT
    pb = (gsel // GPT) * (TILE // 2) + (gsel % GPT)
    pair_ref[...] = pb[:, :, None] + ji[:, 0, :, :] * GPT


def _stage_b_call(gm):
    return pl.pallas_call(
        _stage_b_body,
        grid=(Q // QBLK,),
        in_specs=[pl.BlockSpec((QBLK, NG), lambda i: (i, 0))],
        out_specs=[
            pl.BlockSpec((QBLK, 2, K, NPAIRS), lambda i: (i, 0, 0, 0)),
            pl.BlockSpec((QBLK, K, NPAIRS), lambda i: (i, 0, 0)),
        ],
        out_shape=[
            jax.ShapeDtypeStruct((Q, 2, K, NPAIRS), jnp.int32),
            jax.ShapeDtypeStruct((Q, K, NPAIRS), jnp.int32),
        ],
    )(gm)


def _sc_gather(table, idx_flat, nrows, width, chunk):
    mesh = plsc.VectorSubcoreMesh(core_axis_name="c", subcore_axis_name="s")
    per_w = nrows // SC_NW
    nchunk = per_w // chunk

    @functools.partial(
        pl.kernel, mesh=mesh,
        out_type=jax.ShapeDtypeStruct((nrows, width), jnp.float32),
        scratch_types=[
            pltpu.VMEM((chunk,), jnp.int32),
            pltpu.VMEM((chunk, width), jnp.float32),
            pltpu.SemaphoreType.DMA,
        ],
    )
    def k(table_hbm, idx_hbm, out_hbm, idx_v, rows_v, sem):
        wid = lax.axis_index("s") * SC_NC + lax.axis_index("c")
        for cc in range(nchunk):
            base = wid * per_w + cc * chunk
            pltpu.sync_copy(idx_hbm.at[pl.ds(base, chunk)], idx_v)
            pltpu.async_copy(table_hbm.at[idx_v], rows_v, sem).wait()
            pltpu.sync_copy(rows_v, out_hbm.at[pl.ds(base, chunk)])

    return k(table, idx_flat)


def _sc_gather_elems(table, idx_flat, nelem):
    mesh = plsc.VectorSubcoreMesh(core_axis_name="c", subcore_axis_name="s")
    per_w = nelem // SC_NW

    @functools.partial(
        pl.kernel, mesh=mesh,
        out_type=jax.ShapeDtypeStruct((nelem,), jnp.float32),
        scratch_types=[
            pltpu.VMEM((per_w,), jnp.int32),
            pltpu.VMEM((per_w,), jnp.float32),
            pltpu.SemaphoreType.DMA,
        ],
    )
    def k(table_hbm, idx_hbm, out_hbm, idx_v, out_v, sem):
        wid = lax.axis_index("s") * SC_NC + lax.axis_index("c")
        base = wid * per_w
        pltpu.sync_copy(idx_hbm.at[pl.ds(base, per_w)], idx_v)
        pltpu.async_copy(table_hbm.at[idx_v], out_v, sem).wait()
        pltpu.sync_copy(out_v, out_hbm.at[pl.ds(base, per_w)])

    return k(table, idx_flat)


def _stage_d_body(rows_ref, wc_ref, q_ref, cand_ref, attn_ref, idx_ref):
    rows = rows_ref[...]
    qv = q_ref[...]
    qi = lax.broadcasted_iota(jnp.int32, (QBLK, QBLK, PROWS), 0)
    oi = lax.broadcasted_iota(jnp.int32, (QBLK, QBLK, PROWS), 1)
    halves = []
    for e in range(2):
        kc = rows[:, e * KD:(e + 1) * KD]
        s_all = lax.dot_general(qv, kc, (((1,), (1,)), ((), ())),
                                precision=lax.Precision.DEFAULT)
        s3 = s_all.reshape(QBLK, QBLK, PROWS)
        halves.append(jnp.sum(jnp.where(qi == oi, s3, 0.0), axis=1))
    s = jnp.concatenate(halves, axis=1)

    w = wc_ref[...]
    s = s * w + jnp.where(w == 0.0, NEG, 0.0)

    cand = cand_ref[...]
    sid = lax.broadcasted_iota(jnp.int32, (QBLK, NCAND), 1)
    vs, gs = [], []
    for _ in range(K):
        m = jnp.max(s, axis=1, keepdims=True)
        am = jnp.min(jnp.where(s == m, sid, NCAND), axis=1, keepdims=True)
        gi = jnp.max(jnp.where(sid == am, cand, -1), axis=1, keepdims=True)
        s = jnp.where(sid == am, NEG_INIT, s)
        vs.append(m)
        gs.append(gi)
    vals = jnp.concatenate(vs, axis=1)
    mm = jnp.max(vals, axis=1, keepdims=True)
    e = jnp.exp(vals - mm)
    attn_ref[...] = e / jnp.sum(e, axis=1, keepdims=True)
    idx_ref[...] = jnp.concatenate(gs, axis=1)


def _stage_d_call(rows, w_cand, query_keys, cand_idx):
    nblk = Q // QBLK
    return pl.pallas_call(
        _stage_d_body,
        grid=(nblk,),
        in_specs=[
            pl.BlockSpec((QBLK * PROWS, AUG), lambda i: (i, 0)),
            pl.BlockSpec((QBLK, NCAND), lambda i: (i, 0)),
            pl.BlockSpec((QBLK, KD), lambda i: (i, 0)),
            pl.BlockSpec((QBLK, NCAND), lambda i: (i, 0)),
        ],
        out_specs=[
            pl.BlockSpec((QBLK, K), lambda i: (i, 0)),
            pl.BlockSpec((QBLK, K), lambda i: (i, 0)),
        ],
        out_shape=[
            jax.ShapeDtypeStruct((Q, K), jnp.float32),
            jax.ShapeDtypeStruct((Q, K), jnp.int32),
        ],
    )(rows, w_cand, query_keys, cand_idx)


def _wsum_body(rows_ref, attn_ref, out_ref):
    attn = attn_ref[...]
    acc = attn[:, 0:1] * rows_ref[pl.ds(0, Q), :]
    for kk in range(1, K):
        acc = acc + attn[:, kk:kk + 1] * rows_ref[pl.ds(kk * Q, Q), :]
    out_ref[...] = acc


def _wsum_call(rows, attn):
    return pl.pallas_call(
        _wsum_body,
        out_shape=jax.ShapeDtypeStruct((Q, D), jnp.float32),
    )(rows, attn)


def kernel(memory_bank, temporal_keys, access_counts, last_updated,
           attention_weights, new_patterns, new_keys, query_keys,
           current_time, num_retrievals):
    ct = current_time[0]
    decay = jnp.exp(-(ct - last_updated) * 0.1)
    ti = 0.4 * decay + 0.3 * jnp.log1p(access_counts) + 0.3 * attention_weights
    _, repl_idx = lax.top_k(-ti, B)
    repl_mask = jnp.zeros((M,), jnp.float32).at[repl_idx].set(1.0)
    c_new = 0.4 + 0.3 * jnp.log1p(1.0) + 0.3

    wcol = jnp.concatenate([ti, jnp.full((B,), c_new, jnp.float32)])
    bias = jnp.concatenate([repl_mask * NEG, jnp.zeros((B,), jnp.float32)])
    pad = PCOL - NCOL
    wcol = jnp.pad(wcol, (0, pad))
    bias = jnp.pad(bias, (0, pad), constant_values=NEG)
    keys_ext = jnp.pad(jnp.concatenate([temporal_keys, new_keys], axis=0),
                       ((0, pad), (0, 0)))

    gm = _stage_a_call(query_keys, keys_ext,
                       wcol.reshape(NTILES, 1, TILE),
                       bias.reshape(NTILES, 1, TILE))
    cand, pair = _stage_b_call(gm)
    cand_idx = cand.reshape(Q, NCAND)

    ktab2 = jnp.transpose(
        keys_ext.reshape(NTILES, NPAIRS, 2, GPT, KD),
        (0, 1, 3, 2, 4)).reshape(PCOL // 2, 2 * KD)
    crows = _sc_gather(ktab2, pair.reshape(-1), Q * PROWS, AUG, 512)

    wm = jnp.where(bias < 0.0, 0.0, wcol)
    w_cand = _sc_gather_elems(wm, cand_idx.reshape(-1),
                              Q * NCAND).reshape(Q, NCAND)

    attn, idx16 = _stage_d_call(crows, w_cand, query_keys, cand_idx)

    bank_ext = jnp.concatenate([memory_bank, new_patterns], axis=0)
    idx_flat = idx16.T.reshape(-1)
    rows = _sc_gather(bank_ext, idx_flat, Q * K, D, 512)
    out = _wsum_call(rows, attn)
    del num_retrievals
    return out
